# Initial kernel scaffold; baseline (speedup 1.0000x reference)
import jax
import jax.numpy as jnp
from jax.experimental import pallas as pl


def kernel(x, edge_index, W1, b1, W2, b2):
    raise NotImplementedError("write your pallas kernel here")


# speedup vs baseline: 11.0360x; 11.0360x over previous
import functools

import jax
import jax.numpy as jnp
from jax import lax
from jax.experimental import pallas as pl
from jax.experimental.pallas import tpu as pltpu
from jax.experimental.pallas import tpu_sc as plsc

N_NODES = 10000
N_EDGES = 320000
D_IN = 128
D_HID = 128
N_CLASSES = 40

NCORES = 2
NSUB = 16
NW = NCORES * NSUB
CHUNK = 128
E_PAD = NW * 80 * CHUNK
E_ROWS = E_PAD // CHUNK
ROWS_PER_TILE = E_ROWS // NW
N_PAD = 10240
DSC = 64
DCLS = 64


def _tile_id():
  c = lax.axis_index("c")
  s = lax.axis_index("s")
  return c, s


@functools.lru_cache(maxsize=None)
def _mesh():
  return plsc.VectorSubcoreMesh(
      core_axis_name="c", subcore_axis_name="s",
      num_cores=NCORES, num_subcores=NSUB)


def _deg_body(dst_hbm, out_hbm, idx_v, ones_v, zb_v, acc_sh):
  c, s = _tile_id()
  w = c * NSUB + s

  def fill_ones(i, _):
    ones_v[pl.ds(i * 16, 16)] = jnp.ones((16,), jnp.float32)
    return 0
  lax.fori_loop(0, CHUNK // 16, fill_ones, 0)

  def fill_zb(i, _):
    zb_v[pl.ds(i * 16, 16)] = jnp.zeros((16,), jnp.float32)
    return 0
  lax.fori_loop(0, (N_PAD // NSUB) // 16, fill_zb, 0)

  pltpu.sync_copy(zb_v, acc_sh.at[pl.ds(s * (N_PAD // NSUB), N_PAD // NSUB)])
  plsc.subcore_barrier()

  def body(j, _):
    pltpu.sync_copy(dst_hbm.at[w * ROWS_PER_TILE + j], idx_v)
    pltpu.sync_copy(ones_v, acc_sh.at[idx_v], add=True)
    return 0
  lax.fori_loop(0, ROWS_PER_TILE, body, 0)
  plsc.subcore_barrier()

  @pl.when(s == 0)
  def _():
    pltpu.sync_copy(acc_sh, out_hbm.at[c])


@functools.lru_cache(maxsize=None)
def _deg_call():
  return pl.kernel(
      _deg_body,
      out_type=jax.ShapeDtypeStruct((NCORES, N_PAD), jnp.float32),
      mesh=_mesh(),
      scratch_types=[
          pltpu.VMEM((CHUNK,), jnp.int32),
          pltpu.VMEM((CHUNK,), jnp.float32),
          pltpu.VMEM((N_PAD // NSUB,), jnp.float32),
          pltpu.VMEM_SHARED((N_PAD,), jnp.float32),
      ],
  )


def _scatter_body(zin_hbm, src_hbm, dst_hbm, g_hbm, out_hbm,
                  idxs_v, idxd_v, rows0_v, rows1_v, acc_sh, sem0, sem1):
  c, s = _tile_id()
  w = c * NSUB + s

  @pl.when(s == 0)
  def _():
    pltpu.sync_copy(zin_hbm, acc_sh)

  pltpu.sync_copy(src_hbm.at[pl.ds(w * ROWS_PER_TILE, ROWS_PER_TILE)], idxs_v)
  pltpu.sync_copy(dst_hbm.at[pl.ds(w * ROWS_PER_TILE, ROWS_PER_TILE)], idxd_v)
  plsc.subcore_barrier()

  pltpu.async_copy(g_hbm.at[idxs_v.at[0]], rows0_v, sem0)
  pltpu.async_copy(g_hbm.at[idxs_v.at[1]], rows1_v, sem1)

  def body(k, _):
    j = 2 * k
    pltpu.make_async_copy(g_hbm.at[idxs_v.at[j]], rows0_v, sem0).wait()
    pltpu.sync_copy(rows0_v, acc_sh.at[idxd_v.at[j]], add=True)

    @pl.when(j + 2 < ROWS_PER_TILE)
    def _():
      pltpu.async_copy(g_hbm.at[idxs_v.at[j + 2]], rows0_v, sem0)

    pltpu.make_async_copy(g_hbm.at[idxs_v.at[j + 1]], rows1_v, sem1).wait()
    pltpu.sync_copy(rows1_v, acc_sh.at[idxd_v.at[j + 1]], add=True)

    @pl.when(j + 3 < ROWS_PER_TILE)
    def _():
      pltpu.async_copy(g_hbm.at[idxs_v.at[j + 3]], rows1_v, sem1)
    return 0

  lax.fori_loop(0, ROWS_PER_TILE // 2, body, 0)
  plsc.subcore_barrier()

  @pl.when(s == 0)
  def _():
    pltpu.sync_copy(acc_sh, out_hbm.at[c])


@functools.lru_cache(maxsize=None)
def _scatter_call():
  return pl.kernel(
      _scatter_body,
      out_type=jax.ShapeDtypeStruct((NCORES, N_PAD, DSC), jnp.float32),
      mesh=_mesh(),
      scratch_types=[
          pltpu.VMEM((ROWS_PER_TILE, CHUNK), jnp.int32),
          pltpu.VMEM((ROWS_PER_TILE, CHUNK), jnp.int32),
          pltpu.VMEM((CHUNK, DSC), jnp.float32),
          pltpu.VMEM((CHUNK, DSC), jnp.float32),
          pltpu.VMEM_SHARED((N_PAD, DSC), jnp.float32),
          pltpu.SemaphoreType.DMA,
          pltpu.SemaphoreType.DMA,
      ],
      compiler_params=pltpu.CompilerParams(use_tc_tiling_on_sc=False),
  )


_BLK = 1000
_GRID = N_NODES // _BLK


def _dinv(deg_ref):
  return lax.rsqrt(deg_ref[0] + deg_ref[1] + 1.0)


def _tc1_body(deg_ref, x_ref, w1_ref, g1a_ref, g1b_ref):
  dinv = _dinv(deg_ref)
  h = jnp.dot(x_ref[...], w1_ref[...], preferred_element_type=jnp.float32)
  g1 = h * dinv
  g1a_ref[...] = g1[:, :DSC]
  g1b_ref[...] = g1[:, DSC:]


def _tc2_body(deg_ref, pa_ref, pb_ref, g1a_ref, g1b_ref, w2_ref, b1_ref,
              g2_ref):
  dinv = _dinv(deg_ref)
  agg = jnp.concatenate(
      [pa_ref[0] + pa_ref[1] + g1a_ref[...],
       pb_ref[0] + pb_ref[1] + g1b_ref[...]], axis=1)
  h = jnp.maximum(agg * dinv + b1_ref[...], 0.0)
  g2_ref[...] = jnp.dot(h, w2_ref[...], preferred_element_type=jnp.float32) * dinv


def _tc3_body(deg_ref, p_ref, g2_ref, b2_ref, out_ref):
  dinv = _dinv(deg_ref)
  t = (p_ref[0] + p_ref[1] + g2_ref[...]) * dinv + b2_ref[...]
  col = lax.broadcasted_iota(jnp.int32, (_BLK, DCLS), 1)
  valid = col < N_CLASSES
  tm = jnp.where(valid, t, -1e30)
  m = jnp.max(tm, axis=1, keepdims=True)
  e = jnp.where(valid, jnp.exp(tm - m), 0.0)
  ssum = jnp.sum(e, axis=1, keepdims=True)
  out_ref[...] = (t - m) - jnp.log(ssum)


def _row_spec(d):
  return pl.BlockSpec((_BLK, d), lambda i: (i, 0))


def _deg_spec():
  return pl.BlockSpec((NCORES, _BLK, 1), lambda i: (0, i, 0))


def _part_spec(d):
  return pl.BlockSpec((NCORES, _BLK, d), lambda i: (0, i, 0))


def _full_spec(a, b):
  return pl.BlockSpec((a, b), lambda i: (0, 0))


_tc1_call = pl.pallas_call(
    _tc1_body,
    grid=(_GRID,),
    in_specs=[_deg_spec(), _row_spec(D_IN), _full_spec(D_IN, D_HID)],
    out_specs=[_row_spec(DSC), _row_spec(DSC)],
    out_shape=[jax.ShapeDtypeStruct((N_NODES, DSC), jnp.float32),
               jax.ShapeDtypeStruct((N_NODES, DSC), jnp.float32)],
)

_tc2_call = pl.pallas_call(
    _tc2_body,
    grid=(_GRID,),
    in_specs=[_deg_spec(), _part_spec(DSC), _part_spec(DSC),
              _row_spec(DSC), _row_spec(DSC),
              _full_spec(D_HID, DCLS), _full_spec(1, D_HID)],
    out_specs=_row_spec(DCLS),
    out_shape=jax.ShapeDtypeStruct((N_NODES, DCLS), jnp.float32),
)

_tc3_call = pl.pallas_call(
    _tc3_body,
    grid=(_GRID,),
    in_specs=[_deg_spec(), _part_spec(DCLS), _row_spec(DCLS),
              _full_spec(1, DCLS)],
    out_specs=_row_spec(DCLS),
    out_shape=jax.ShapeDtypeStruct((N_NODES, DCLS), jnp.float32),
)


@jax.jit
def kernel(x, edge_index, W1, b1, W2, b2):
  src = edge_index[0].astype(jnp.int32)
  dst = edge_index[1].astype(jnp.int32)
  pad = E_PAD - N_EDGES
  src2d = jnp.concatenate([src, jnp.zeros((pad,), jnp.int32)]).reshape(E_ROWS, CHUNK)
  dst2d = jnp.concatenate(
      [dst, jnp.full((pad,), N_NODES, jnp.int32)]).reshape(E_ROWS, CHUNK)

  W2p = jnp.zeros((D_HID, DCLS), jnp.float32).at[:, :N_CLASSES].set(W2)
  b1r = b1.reshape(1, D_HID)
  b2r = jnp.zeros((1, DCLS), jnp.float32).at[0, :N_CLASSES].set(b2)

  zin = jnp.zeros((N_PAD, DSC), jnp.float32)

  deg_part = _deg_call()(dst2d)
  deg3 = deg_part.reshape(NCORES, N_PAD, 1)

  g1a, g1b = _tc1_call(deg3, x, W1)
  scat = _scatter_call()
  pa = scat(zin, src2d, dst2d, g1a)
  pb = scat(zin, src2d, dst2d, g1b)
  g2 = _tc2_call(deg3, pa, pb, g1a, g1b, W2p, b1r)
  p2 = scat(zin, src2d, dst2d, g2)
  out = _tc3_call(deg3, p2, g2, b2r)
  return out[:, :N_CLASSES]


# speedup vs baseline: 11.3742x; 1.0306x over previous
import functools

import jax
import jax.numpy as jnp
from jax import lax
from jax.experimental import pallas as pl
from jax.experimental.pallas import tpu as pltpu
from jax.experimental.pallas import tpu_sc as plsc

N_NODES = 10000
N_EDGES = 320000
D_IN = 128
D_HID = 128
N_CLASSES = 40

NCORES = 2
NSUB = 16
NW = NCORES * NSUB
CHUNK = 128
E_PAD = NW * 80 * CHUNK
E_ROWS = E_PAD // CHUNK
ROWS_PER_TILE = E_ROWS // NW
N_PAD = 10240
DSC = 64
DCLS = 64


def _tile_id():
  c = lax.axis_index("c")
  s = lax.axis_index("s")
  return c, s


@functools.lru_cache(maxsize=None)
def _mesh():
  return plsc.VectorSubcoreMesh(
      core_axis_name="c", subcore_axis_name="s",
      num_cores=NCORES, num_subcores=NSUB)


def _deg_body(dst_hbm, out_hbm, idx_v, ones_v, zb_v, acc_sh, sem):
  c, s = _tile_id()
  w = c * NSUB + s

  def fill_ones(i, _):
    ones_v[pl.ds(i * 16, 16)] = jnp.ones((16,), jnp.float32)
    return 0
  lax.fori_loop(0, CHUNK // 16, fill_ones, 0)

  def fill_zb(i, _):
    zb_v[pl.ds(i * 16, 16)] = jnp.zeros((16,), jnp.float32)
    return 0
  lax.fori_loop(0, (N_PAD // NSUB) // 16, fill_zb, 0)

  pltpu.sync_copy(zb_v, acc_sh.at[pl.ds(s * (N_PAD // NSUB), N_PAD // NSUB)])
  pltpu.sync_copy(dst_hbm.at[pl.ds(w * ROWS_PER_TILE, ROWS_PER_TILE)], idx_v)
  plsc.subcore_barrier()

  def body(j, _):
    pltpu.async_copy(ones_v, acc_sh.at[idx_v.at[j]], sem, add=True)
    return 0
  lax.fori_loop(0, ROWS_PER_TILE, body, 0)

  def drain(j, _):
    pltpu.make_async_copy(ones_v, acc_sh.at[idx_v.at[j]], sem).wait()
    return 0
  lax.fori_loop(0, ROWS_PER_TILE, drain, 0)
  plsc.subcore_barrier()

  @pl.when(s == 0)
  def _():
    pltpu.sync_copy(acc_sh, out_hbm.at[c])


@functools.lru_cache(maxsize=None)
def _deg_call():
  return pl.kernel(
      _deg_body,
      out_type=jax.ShapeDtypeStruct((NCORES, N_PAD), jnp.float32),
      mesh=_mesh(),
      scratch_types=[
          pltpu.VMEM((ROWS_PER_TILE, CHUNK), jnp.int32),
          pltpu.VMEM((CHUNK,), jnp.float32),
          pltpu.VMEM((N_PAD // NSUB,), jnp.float32),
          pltpu.VMEM_SHARED((N_PAD,), jnp.float32),
          pltpu.SemaphoreType.DMA,
      ],
  )


NBUF = 8
PREF = 4


def _scatter_body(zin_hbm, src_hbm, dst_hbm, g_hbm, out_hbm,
                  idxs_v, idxd_v, r0, r1, r2, r3, r4, r5, r6, r7,
                  acc_sh, semg, sems):
  rows = (r0, r1, r2, r3, r4, r5, r6, r7)
  c, s = _tile_id()
  w = c * NSUB + s

  @pl.when(s == 0)
  def _():
    pltpu.sync_copy(zin_hbm, acc_sh)

  pltpu.sync_copy(src_hbm.at[pl.ds(w * ROWS_PER_TILE, ROWS_PER_TILE)], idxs_v)
  pltpu.sync_copy(dst_hbm.at[pl.ds(w * ROWS_PER_TILE, ROWS_PER_TILE)], idxd_v)
  plsc.subcore_barrier()

  for b in range(PREF):
    pltpu.async_copy(g_hbm.at[idxs_v.at[b]], rows[b], semg.at[b])

  def body(k, _):
    for b in range(NBUF):
      j = k * NBUF + b

      @pl.when(j >= PREF)
      def _():
        jo = j - PREF
        pltpu.make_async_copy(
            rows[(b - PREF) % NBUF], acc_sh.at[idxd_v.at[jo]],
            sems.at[(b - PREF) % NBUF]).wait()

      @pl.when(j + PREF < ROWS_PER_TILE)
      def _():
        jn = j + PREF
        pltpu.async_copy(g_hbm.at[idxs_v.at[jn]], rows[(b + PREF) % NBUF],
                         semg.at[(b + PREF) % NBUF])

      pltpu.make_async_copy(g_hbm.at[idxs_v.at[j]], rows[b], semg.at[b]).wait()
      pltpu.async_copy(rows[b], acc_sh.at[idxd_v.at[j]], sems.at[b], add=True)
    return 0

  lax.fori_loop(0, ROWS_PER_TILE // NBUF, body, 0)
  for b in range(PREF):
    j = ROWS_PER_TILE - PREF + b
    pltpu.make_async_copy(rows[j % NBUF], acc_sh.at[idxd_v.at[j]],
                          sems.at[j % NBUF]).wait()
  plsc.subcore_barrier()

  @pl.when(s == 0)
  def _():
    pltpu.sync_copy(acc_sh, out_hbm.at[c])


@functools.lru_cache(maxsize=None)
def _scatter_call():
  return pl.kernel(
      _scatter_body,
      out_type=jax.ShapeDtypeStruct((NCORES, N_PAD, DSC), jnp.float32),
      mesh=_mesh(),
      scratch_types=(
          [pltpu.VMEM((ROWS_PER_TILE, CHUNK), jnp.int32),
           pltpu.VMEM((ROWS_PER_TILE, CHUNK), jnp.int32)]
          + [pltpu.VMEM((CHUNK, DSC), jnp.float32) for _ in range(NBUF)]
          + [pltpu.VMEM_SHARED((N_PAD, DSC), jnp.float32),
             pltpu.SemaphoreType.DMA((NBUF,)),
             pltpu.SemaphoreType.DMA((NBUF,))]
      ),
      compiler_params=pltpu.CompilerParams(use_tc_tiling_on_sc=False),
  )


_BLK = 1000
_GRID = N_NODES // _BLK


def _dinv(deg_ref):
  return lax.rsqrt(deg_ref[0] + deg_ref[1] + 1.0)


def _tc1_body(deg_ref, x_ref, w1_ref, g1a_ref, g1b_ref):
  dinv = _dinv(deg_ref)
  h = jnp.dot(x_ref[...], w1_ref[...], preferred_element_type=jnp.float32)
  g1 = h * dinv
  g1a_ref[...] = g1[:, :DSC]
  g1b_ref[...] = g1[:, DSC:]


def _tc2_body(deg_ref, pa_ref, pb_ref, g1a_ref, g1b_ref, w2_ref, b1_ref,
              g2_ref):
  dinv = _dinv(deg_ref)
  agg = jnp.concatenate(
      [pa_ref[0] + pa_ref[1] + g1a_ref[...],
       pb_ref[0] + pb_ref[1] + g1b_ref[...]], axis=1)
  h = jnp.maximum(agg * dinv + b1_ref[...], 0.0)
  g2_ref[...] = jnp.dot(h, w2_ref[...], preferred_element_type=jnp.float32) * dinv


def _tc3_body(deg_ref, p_ref, g2_ref, b2_ref, out_ref):
  dinv = _dinv(deg_ref)
  t = (p_ref[0] + p_ref[1] + g2_ref[...]) * dinv + b2_ref[...]
  col = lax.broadcasted_iota(jnp.int32, (_BLK, DCLS), 1)
  valid = col < N_CLASSES
  tm = jnp.where(valid, t, -1e30)
  m = jnp.max(tm, axis=1, keepdims=True)
  e = jnp.where(valid, jnp.exp(tm - m), 0.0)
  ssum = jnp.sum(e, axis=1, keepdims=True)
  out_ref[...] = (t - m) - jnp.log(ssum)


def _row_spec(d):
  return pl.BlockSpec((_BLK, d), lambda i: (i, 0))


def _deg_spec():
  return pl.BlockSpec((NCORES, _BLK, 1), lambda i: (0, i, 0))


def _part_spec(d):
  return pl.BlockSpec((NCORES, _BLK, d), lambda i: (0, i, 0))


def _full_spec(a, b):
  return pl.BlockSpec((a, b), lambda i: (0, 0))


_tc1_call = pl.pallas_call(
    _tc1_body,
    grid=(_GRID,),
    in_specs=[_deg_spec(), _row_spec(D_IN), _full_spec(D_IN, D_HID)],
    out_specs=[_row_spec(DSC), _row_spec(DSC)],
    out_shape=[jax.ShapeDtypeStruct((N_NODES, DSC), jnp.float32),
               jax.ShapeDtypeStruct((N_NODES, DSC), jnp.float32)],
)

_tc2_call = pl.pallas_call(
    _tc2_body,
    grid=(_GRID,),
    in_specs=[_deg_spec(), _part_spec(DSC), _part_spec(DSC),
              _row_spec(DSC), _row_spec(DSC),
              _full_spec(D_HID, DCLS), _full_spec(1, D_HID)],
    out_specs=_row_spec(DCLS),
    out_shape=jax.ShapeDtypeStruct((N_NODES, DCLS), jnp.float32),
)

_tc3_call = pl.pallas_call(
    _tc3_body,
    grid=(_GRID,),
    in_specs=[_deg_spec(), _part_spec(DCLS), _row_spec(DCLS),
              _full_spec(1, DCLS)],
    out_specs=_row_spec(DCLS),
    out_shape=jax.ShapeDtypeStruct((N_NODES, DCLS), jnp.float32),
)


@jax.jit
def kernel(x, edge_index, W1, b1, W2, b2):
  src = edge_index[0].astype(jnp.int32)
  dst = edge_index[1].astype(jnp.int32)
  pad = E_PAD - N_EDGES
  src2d = jnp.concatenate([src, jnp.zeros((pad,), jnp.int32)]).reshape(E_ROWS, CHUNK)
  dst2d = jnp.concatenate(
      [dst, jnp.full((pad,), N_NODES, jnp.int32)]).reshape(E_ROWS, CHUNK)

  W2p = jnp.zeros((D_HID, DCLS), jnp.float32).at[:, :N_CLASSES].set(W2)
  b1r = b1.reshape(1, D_HID)
  b2r = jnp.zeros((1, DCLS), jnp.float32).at[0, :N_CLASSES].set(b2)

  zin = jnp.zeros((N_PAD, DSC), jnp.float32)

  deg_part = _deg_call()(dst2d)
  deg3 = deg_part.reshape(NCORES, N_PAD, 1)

  g1a, g1b = _tc1_call(deg3, x, W1)
  scat = _scatter_call()
  pa = scat(zin, src2d, dst2d, g1a)
  pb = scat(zin, src2d, dst2d, g1b)
  g2 = _tc2_call(deg3, pa, pb, g1a, g1b, W2p, b1r)
  p2 = scat(zin, src2d, dst2d, g2)
  out = _tc3_call(deg3, p2, g2, b2r)
  return out[:, :N_CLASSES]


# speedup vs baseline: 12.6694x; 1.1139x over previous
import functools

import jax
import jax.numpy as jnp
from jax import lax
from jax.experimental import pallas as pl
from jax.experimental.pallas import tpu as pltpu
from jax.experimental.pallas import tpu_sc as plsc

N_NODES = 10000
N_EDGES = 320000
D_IN = 128
D_HID = 128
N_CLASSES = 40

NCORES = 2
NSUB = 16
NW = NCORES * NSUB
CHUNK = 128
E_PAD = NW * 80 * CHUNK
E_ROWS = E_PAD // CHUNK
ROWS_PER_TILE = E_ROWS // NW
N_PAD = 10240
DSC = 64
DCLS = 64


def _tile_id():
  c = lax.axis_index("c")
  s = lax.axis_index("s")
  return c, s


@functools.lru_cache(maxsize=None)
def _mesh():
  return plsc.VectorSubcoreMesh(
      core_axis_name="c", subcore_axis_name="s",
      num_cores=NCORES, num_subcores=NSUB)


def _deg_body(dst_hbm, out_hbm, idx_v, ones_v, zb_v, acc_sh, sem):
  c, s = _tile_id()
  w = c * NSUB + s

  def fill_ones(i, _):
    ones_v[pl.ds(i * 16, 16)] = jnp.ones((16,), jnp.float32)
    return 0
  lax.fori_loop(0, CHUNK // 16, fill_ones, 0)

  def fill_zb(i, _):
    zb_v[pl.ds(i * 16, 16)] = jnp.zeros((16,), jnp.float32)
    return 0
  lax.fori_loop(0, (N_PAD // NSUB) // 16, fill_zb, 0)

  pltpu.sync_copy(zb_v, acc_sh.at[pl.ds(s * (N_PAD // NSUB), N_PAD // NSUB)])
  pltpu.sync_copy(dst_hbm.at[pl.ds(w * ROWS_PER_TILE, ROWS_PER_TILE)], idx_v)
  plsc.subcore_barrier()

  def body(j, _):
    pltpu.async_copy(ones_v, acc_sh.at[idx_v.at[j]], sem, add=True)
    return 0
  lax.fori_loop(0, ROWS_PER_TILE, body, 0)

  def drain(j, _):
    pltpu.make_async_copy(ones_v, acc_sh.at[idx_v.at[j]], sem).wait()
    return 0
  lax.fori_loop(0, ROWS_PER_TILE, drain, 0)
  plsc.subcore_barrier()

  @pl.when(s == 0)
  def _():
    pltpu.sync_copy(acc_sh, out_hbm.at[c])


@functools.lru_cache(maxsize=None)
def _deg_call():
  return pl.kernel(
      _deg_body,
      out_type=jax.ShapeDtypeStruct((NCORES, N_PAD), jnp.float32),
      mesh=_mesh(),
      scratch_types=[
          pltpu.VMEM((ROWS_PER_TILE, CHUNK), jnp.int32),
          pltpu.VMEM((CHUNK,), jnp.float32),
          pltpu.VMEM((N_PAD // NSUB,), jnp.float32),
          pltpu.VMEM_SHARED((N_PAD,), jnp.float32),
          pltpu.SemaphoreType.DMA,
      ],
  )


NBUF = 8
PREF = 4


def _scatter_body(zin_hbm, src_hbm, dst_hbm, g_hbm, out_hbm,
                  idxs_v, idxd_v, r0, r1, r2, r3, r4, r5, r6, r7,
                  acc_sh, semg, sems):
  rows = (r0, r1, r2, r3, r4, r5, r6, r7)
  c, s = _tile_id()
  w = c * NSUB + s

  @pl.when(s == 0)
  def _():
    pltpu.sync_copy(zin_hbm, acc_sh)

  pltpu.sync_copy(src_hbm.at[pl.ds(w * ROWS_PER_TILE, ROWS_PER_TILE)], idxs_v)
  pltpu.sync_copy(dst_hbm.at[pl.ds(w * ROWS_PER_TILE, ROWS_PER_TILE)], idxd_v)
  plsc.subcore_barrier()

  for b in range(PREF):
    pltpu.async_copy(g_hbm.at[idxs_v.at[b]], rows[b], semg.at[b])

  def body(k, _):
    for b in range(NBUF):
      j = k * NBUF + b

      @pl.when(j >= PREF)
      def _():
        jo = j - PREF
        pltpu.make_async_copy(
            rows[(b - PREF) % NBUF], acc_sh.at[idxd_v.at[jo]],
            sems.at[(b - PREF) % NBUF]).wait()

      @pl.when(j + PREF < ROWS_PER_TILE)
      def _():
        jn = j + PREF
        pltpu.async_copy(g_hbm.at[idxs_v.at[jn]], rows[(b + PREF) % NBUF],
                         semg.at[(b + PREF) % NBUF])

      pltpu.make_async_copy(g_hbm.at[idxs_v.at[j]], rows[b], semg.at[b]).wait()
      pltpu.async_copy(rows[b], acc_sh.at[idxd_v.at[j]], sems.at[b], add=True)
    return 0

  lax.fori_loop(0, ROWS_PER_TILE // NBUF, body, 0)
  for b in range(PREF):
    j = ROWS_PER_TILE - PREF + b
    pltpu.make_async_copy(rows[j % NBUF], acc_sh.at[idxd_v.at[j]],
                          sems.at[j % NBUF]).wait()
  plsc.subcore_barrier()

  @pl.when(s == 0)
  def _():
    pltpu.sync_copy(acc_sh, out_hbm.at[c])


@functools.lru_cache(maxsize=None)
def _scatter_call():
  return pl.kernel(
      _scatter_body,
      out_type=jax.ShapeDtypeStruct((NCORES, N_PAD, DSC), jnp.float32),
      mesh=_mesh(),
      scratch_types=(
          [pltpu.VMEM((ROWS_PER_TILE, CHUNK), jnp.int32),
           pltpu.VMEM((ROWS_PER_TILE, CHUNK), jnp.int32)]
          + [pltpu.VMEM((CHUNK, DSC), jnp.float32) for _ in range(NBUF)]
          + [pltpu.VMEM_SHARED((N_PAD, DSC), jnp.float32),
             pltpu.SemaphoreType.DMA((NBUF,)),
             pltpu.SemaphoreType.DMA((NBUF,))]
      ),
      compiler_params=pltpu.CompilerParams(use_tc_tiling_on_sc=False),
  )


_BLK = 1000
_GRID = N_NODES // _BLK


def _dinv(deg_ref):
  return lax.rsqrt(deg_ref[0] + deg_ref[1] + 1.0)


def _tc1_body(deg_ref, x_ref, w1_ref, g1a_ref, g1b_ref):
  dinv = _dinv(deg_ref)
  h = jnp.dot(x_ref[...], w1_ref[...], preferred_element_type=jnp.float32)
  g1 = h * dinv
  g1a_ref[...] = g1[:, :DSC]
  g1b_ref[...] = g1[:, DSC:]


def _tc2_body(deg_ref, pa_ref, pb_ref, g1a_ref, g1b_ref, w2_ref, b1_ref,
              g2_ref):
  dinv = _dinv(deg_ref)
  agg = jnp.concatenate(
      [pa_ref[0] + pa_ref[1] + g1a_ref[...],
       pb_ref[0] + pb_ref[1] + g1b_ref[...]], axis=1)
  h = jnp.maximum(agg * dinv + b1_ref[...], 0.0)
  g2_ref[...] = jnp.dot(h, w2_ref[...], preferred_element_type=jnp.float32) * dinv


def _tc3_body(deg_ref, p_ref, g2_ref, b2_ref, out_ref):
  dinv = _dinv(deg_ref)
  t = (p_ref[0] + p_ref[1] + g2_ref[...]) * dinv + b2_ref[...]
  col = lax.broadcasted_iota(jnp.int32, (_BLK, DCLS), 1)
  valid = col < N_CLASSES
  tm = jnp.where(valid, t, -1e30)
  m = jnp.max(tm, axis=1, keepdims=True)
  e = jnp.where(valid, jnp.exp(tm - m), 0.0)
  ssum = jnp.sum(e, axis=1, keepdims=True)
  out_ref[...] = (t - m) - jnp.log(ssum)


def _row_spec(d):
  return pl.BlockSpec((_BLK, d), lambda i: (i, 0))


def _deg_spec():
  return pl.BlockSpec((NCORES, _BLK, 1), lambda i: (0, i, 0))


def _part_spec(d):
  return pl.BlockSpec((NCORES, _BLK, d), lambda i: (0, i, 0))


def _full_spec(a, b):
  return pl.BlockSpec((a, b), lambda i: (0, 0))


_tc1_call = pl.pallas_call(
    _tc1_body,
    grid=(_GRID,),
    in_specs=[_deg_spec(), _row_spec(D_IN), _full_spec(D_IN, D_HID)],
    out_specs=[_row_spec(DSC), _row_spec(DSC)],
    out_shape=[jax.ShapeDtypeStruct((N_NODES, DSC), jnp.float32),
               jax.ShapeDtypeStruct((N_NODES, DSC), jnp.float32)],
)

_tc2_call = pl.pallas_call(
    _tc2_body,
    grid=(_GRID,),
    in_specs=[_deg_spec(), _part_spec(DSC), _part_spec(DSC),
              _row_spec(DSC), _row_spec(DSC),
              _full_spec(D_HID, DCLS), _full_spec(1, D_HID)],
    out_specs=_row_spec(DCLS),
    out_shape=jax.ShapeDtypeStruct((N_NODES, DCLS), jnp.float32),
)

_tc3_call = pl.pallas_call(
    _tc3_body,
    grid=(_GRID,),
    in_specs=[_deg_spec(), _part_spec(DCLS), _row_spec(DCLS),
              _full_spec(1, DCLS)],
    out_specs=_row_spec(DCLS),
    out_shape=jax.ShapeDtypeStruct((N_NODES, DCLS), jnp.float32),
)


@jax.jit
def kernel(x, edge_index, W1, b1, W2, b2):
  src = edge_index[0].astype(jnp.int32)
  dst = edge_index[1].astype(jnp.int32)
  pad = E_PAD - N_EDGES
  pad_dst = N_NODES + (jnp.arange(pad, dtype=jnp.int32) % (N_PAD - N_NODES))
  src2d = jnp.concatenate([src, jnp.zeros((pad,), jnp.int32)]).reshape(E_ROWS, CHUNK)
  dst2d = jnp.concatenate([dst, pad_dst]).reshape(E_ROWS, CHUNK)

  W2p = jnp.zeros((D_HID, DCLS), jnp.float32).at[:, :N_CLASSES].set(W2)
  b1r = b1.reshape(1, D_HID)
  b2r = jnp.zeros((1, DCLS), jnp.float32).at[0, :N_CLASSES].set(b2)

  zin = jnp.zeros((N_PAD, DSC), jnp.float32)

  deg_part = _deg_call()(dst2d)
  deg3 = deg_part.reshape(NCORES, N_PAD, 1)

  g1a, g1b = _tc1_call(deg3, x, W1)
  scat = _scatter_call()
  pa = scat(zin, src2d, dst2d, g1a)
  pb = scat(zin, src2d, dst2d, g1b)
  g2 = _tc2_call(deg3, pa, pb, g1a, g1b, W2p, b1r)
  p2 = scat(zin, src2d, dst2d, g2)
  out = _tc3_call(deg3, p2, g2, b2r)
  return out[:, :N_CLASSES]


# speedup vs baseline: 28.7853x; 2.2720x over previous
import functools

import jax
import jax.numpy as jnp
from jax import lax
from jax.experimental import pallas as pl
from jax.experimental.pallas import tpu as pltpu
from jax.experimental.pallas import tpu_sc as plsc

N_NODES = 10000
N_EDGES = 320000
D_IN = 128
D_HID = 128
N_CLASSES = 40

NCORES = 2
NSUB = 16
NW = NCORES * NSUB
CHUNK = 128
E_PAD = NW * 80 * CHUNK
E_ROWS = E_PAD // CHUNK
ROWS_PER_TILE = E_ROWS // NW
N_PAD = 10240
DSC = 64
DCLS = 64


def _tile_id():
  c = lax.axis_index("c")
  s = lax.axis_index("s")
  return c, s


@functools.lru_cache(maxsize=None)
def _mesh():
  return plsc.VectorSubcoreMesh(
      core_axis_name="c", subcore_axis_name="s",
      num_cores=NCORES, num_subcores=NSUB)


def _deg_body(dst_hbm, out_hbm, idx_v, ones_v, zb_v, acc_sh, sem):
  c, s = _tile_id()
  w = c * NSUB + s

  def fill_ones(i, _):
    ones_v[pl.ds(i * 16, 16)] = jnp.ones((16,), jnp.float32)
    return 0
  lax.fori_loop(0, CHUNK // 16, fill_ones, 0)

  def fill_zb(i, _):
    zb_v[pl.ds(i * 16, 16)] = jnp.zeros((16,), jnp.float32)
    return 0
  lax.fori_loop(0, (N_PAD // NSUB) // 16, fill_zb, 0)

  pltpu.sync_copy(zb_v, acc_sh.at[pl.ds(s * (N_PAD // NSUB), N_PAD // NSUB)])
  pltpu.sync_copy(dst_hbm.at[pl.ds(w * ROWS_PER_TILE, ROWS_PER_TILE)], idx_v)
  plsc.subcore_barrier()

  def body(j, _):
    pltpu.async_copy(ones_v, acc_sh.at[idx_v.at[j]], sem, add=True)
    return 0
  lax.fori_loop(0, ROWS_PER_TILE, body, 0)

  def drain(j, _):
    pltpu.make_async_copy(ones_v, acc_sh.at[idx_v.at[j]], sem).wait()
    return 0
  lax.fori_loop(0, ROWS_PER_TILE, drain, 0)
  plsc.subcore_barrier()

  @pl.when(s == 0)
  def _():
    pltpu.sync_copy(acc_sh, out_hbm.at[c])


@functools.lru_cache(maxsize=None)
def _deg_call():
  return pl.kernel(
      _deg_body,
      out_type=jax.ShapeDtypeStruct((NCORES, N_PAD), jnp.float32),
      mesh=_mesh(),
      scratch_types=[
          pltpu.VMEM((ROWS_PER_TILE, CHUNK), jnp.int32),
          pltpu.VMEM((CHUNK,), jnp.float32),
          pltpu.VMEM((N_PAD // NSUB,), jnp.float32),
          pltpu.VMEM_SHARED((N_PAD,), jnp.float32),
          pltpu.SemaphoreType.DMA,
      ],
  )


NBUF = 8
PREF = 4
DHALF = DSC // 2
ROWS_PER_TILE2 = E_ROWS // NSUB


def _scatter_body(zin_hbm, src_hbm, dst_hbm, glo_hbm, ghi_hbm,
                  outlo_hbm, outhi_hbm,
                  idxs_v, idxd_v, r0, r1, r2, r3, r4, r5, r6, r7,
                  acc_sh, g_sh, semg, sems):
  rows = (r0, r1, r2, r3, r4, r5, r6, r7)
  c, s = _tile_id()

  @pl.when(s == 0)
  def _():
    pltpu.sync_copy(zin_hbm, acc_sh)

  @pl.when((s == 1) & (c == 0))
  def _():
    pltpu.sync_copy(glo_hbm, g_sh)

  @pl.when((s == 1) & (c == 1))
  def _():
    pltpu.sync_copy(ghi_hbm, g_sh)

  pltpu.sync_copy(src_hbm.at[pl.ds(s * ROWS_PER_TILE2, ROWS_PER_TILE2)], idxs_v)
  pltpu.sync_copy(dst_hbm.at[pl.ds(s * ROWS_PER_TILE2, ROWS_PER_TILE2)], idxd_v)
  plsc.subcore_barrier()

  for b in range(PREF):
    pltpu.async_copy(g_sh.at[idxs_v.at[b]], rows[b], semg.at[b])

  def body(k, _):
    for b in range(NBUF):
      j = k * NBUF + b

      @pl.when(j >= PREF)
      def _():
        jo = j - PREF
        pltpu.make_async_copy(
            rows[(b - PREF) % NBUF], acc_sh.at[idxd_v.at[jo]],
            sems.at[(b - PREF) % NBUF]).wait()

      @pl.when(j + PREF < ROWS_PER_TILE2)
      def _():
        jn = j + PREF
        pltpu.async_copy(g_sh.at[idxs_v.at[jn]], rows[(b + PREF) % NBUF],
                         semg.at[(b + PREF) % NBUF])

      pltpu.make_async_copy(g_sh.at[idxs_v.at[j]], rows[b], semg.at[b]).wait()
      pltpu.async_copy(rows[b], acc_sh.at[idxd_v.at[j]], sems.at[b], add=True)
    return 0

  lax.fori_loop(0, ROWS_PER_TILE2 // NBUF, body, 0)
  for b in range(PREF):
    j = ROWS_PER_TILE2 - PREF + b
    pltpu.make_async_copy(rows[j % NBUF], acc_sh.at[idxd_v.at[j]],
                          sems.at[j % NBUF]).wait()
  plsc.subcore_barrier()

  @pl.when((s == 0) & (c == 0))
  def _():
    pltpu.sync_copy(acc_sh, outlo_hbm)

  @pl.when((s == 0) & (c == 1))
  def _():
    pltpu.sync_copy(acc_sh, outhi_hbm)


@functools.lru_cache(maxsize=None)
def _scatter_call():
  return pl.kernel(
      _scatter_body,
      out_type=[jax.ShapeDtypeStruct((N_PAD, DHALF), jnp.float32),
                jax.ShapeDtypeStruct((N_PAD, DHALF), jnp.float32)],
      mesh=_mesh(),
      scratch_types=(
          [pltpu.VMEM((ROWS_PER_TILE2, CHUNK), jnp.int32),
           pltpu.VMEM((ROWS_PER_TILE2, CHUNK), jnp.int32)]
          + [pltpu.VMEM((CHUNK, DHALF), jnp.float32) for _ in range(NBUF)]
          + [pltpu.VMEM_SHARED((N_PAD, DHALF), jnp.float32),
             pltpu.VMEM_SHARED((N_NODES, DHALF), jnp.float32),
             pltpu.SemaphoreType.DMA((NBUF,)),
             pltpu.SemaphoreType.DMA((NBUF,))]
      ),
      compiler_params=pltpu.CompilerParams(use_tc_tiling_on_sc=False),
  )


_BLK = 1000
_GRID = N_NODES // _BLK


def _dinv(deg_ref):
  return lax.rsqrt(deg_ref[0] + deg_ref[1] + 1.0)


def _tc1_body(deg_ref, x_ref, w1_ref, g10_ref, g11_ref, g12_ref, g13_ref):
  dinv = _dinv(deg_ref)
  h = jnp.dot(x_ref[...], w1_ref[...], preferred_element_type=jnp.float32)
  g1 = h * dinv
  g10_ref[...] = g1[:, 0 * DHALF:1 * DHALF]
  g11_ref[...] = g1[:, 1 * DHALF:2 * DHALF]
  g12_ref[...] = g1[:, 2 * DHALF:3 * DHALF]
  g13_ref[...] = g1[:, 3 * DHALF:4 * DHALF]


def _tc2_body(deg_ref, a0_ref, a1_ref, a2_ref, a3_ref,
              g10_ref, g11_ref, g12_ref, g13_ref, w2_ref, b1_ref,
              g20_ref, g21_ref):
  dinv = _dinv(deg_ref)
  agg = jnp.concatenate(
      [a0_ref[...] + g10_ref[...], a1_ref[...] + g11_ref[...],
       a2_ref[...] + g12_ref[...], a3_ref[...] + g13_ref[...]], axis=1)
  h = jnp.maximum(agg * dinv + b1_ref[...], 0.0)
  g2 = jnp.dot(h, w2_ref[...], preferred_element_type=jnp.float32) * dinv
  g20_ref[...] = g2[:, :DHALF]
  g21_ref[...] = g2[:, DHALF:]


def _tc3_body(deg_ref, a0_ref, a1_ref, g20_ref, g21_ref, b2_ref, out_ref):
  dinv = _dinv(deg_ref)
  t = jnp.concatenate(
      [a0_ref[...] + g20_ref[...], a1_ref[...] + g21_ref[...]], axis=1)
  t = t * dinv + b2_ref[...]
  col = lax.broadcasted_iota(jnp.int32, (_BLK, DCLS), 1)
  valid = col < N_CLASSES
  tm = jnp.where(valid, t, -1e30)
  m = jnp.max(tm, axis=1, keepdims=True)
  e = jnp.where(valid, jnp.exp(tm - m), 0.0)
  ssum = jnp.sum(e, axis=1, keepdims=True)
  out_ref[...] = (t - m) - jnp.log(ssum)


def _row_spec(d):
  return pl.BlockSpec((_BLK, d), lambda i: (i, 0))


def _deg_spec():
  return pl.BlockSpec((NCORES, _BLK, 1), lambda i: (0, i, 0))


def _full_spec(a, b):
  return pl.BlockSpec((a, b), lambda i: (0, 0))


_tc1_call = pl.pallas_call(
    _tc1_body,
    grid=(_GRID,),
    in_specs=[_deg_spec(), _row_spec(D_IN), _full_spec(D_IN, D_HID)],
    out_specs=[_row_spec(DHALF)] * 4,
    out_shape=[jax.ShapeDtypeStruct((N_NODES, DHALF), jnp.float32)] * 4,
)

_tc2_call = pl.pallas_call(
    _tc2_body,
    grid=(_GRID,),
    in_specs=[_deg_spec()] + [_row_spec(DHALF)] * 8
             + [_full_spec(D_HID, DCLS), _full_spec(1, D_HID)],
    out_specs=[_row_spec(DHALF)] * 2,
    out_shape=[jax.ShapeDtypeStruct((N_NODES, DHALF), jnp.float32)] * 2,
)

_tc3_call = pl.pallas_call(
    _tc3_body,
    grid=(_GRID,),
    in_specs=[_deg_spec()] + [_row_spec(DHALF)] * 4 + [_full_spec(1, DCLS)],
    out_specs=_row_spec(DCLS),
    out_shape=jax.ShapeDtypeStruct((N_NODES, DCLS), jnp.float32),
)


@jax.jit
def kernel(x, edge_index, W1, b1, W2, b2):
  src = edge_index[0].astype(jnp.int32)
  dst = edge_index[1].astype(jnp.int32)
  pad = E_PAD - N_EDGES
  pad_dst = N_NODES + (jnp.arange(pad, dtype=jnp.int32) % (N_PAD - N_NODES))
  src2d = jnp.concatenate([src, jnp.zeros((pad,), jnp.int32)]).reshape(E_ROWS, CHUNK)
  dst2d = jnp.concatenate([dst, pad_dst]).reshape(E_ROWS, CHUNK)

  W2p = jnp.zeros((D_HID, DCLS), jnp.float32).at[:, :N_CLASSES].set(W2)
  b1r = b1.reshape(1, D_HID)
  b2r = jnp.zeros((1, DCLS), jnp.float32).at[0, :N_CLASSES].set(b2)
  zin = jnp.zeros((N_PAD, DHALF), jnp.float32)

  deg_part = _deg_call()(dst2d)
  deg3 = deg_part.reshape(NCORES, N_PAD, 1)

  g10, g11, g12, g13 = _tc1_call(deg3, x, W1)
  scat = _scatter_call()
  a0, a1 = scat(zin, src2d, dst2d, g10, g11)
  a2, a3 = scat(zin, src2d, dst2d, g12, g13)
  g20, g21 = _tc2_call(deg3, a0, a1, a2, a3,
                       g10, g11, g12, g13, W2p, b1r)
  b0, b1_ = scat(zin, src2d, dst2d, g20, g21)
  out = _tc3_call(deg3, b0, b1_, g20, g21, b2r)
  return out[:, :N_CLASSES]


# speedup vs baseline: 28.8229x; 1.0013x over previous
import functools

import jax
import jax.numpy as jnp
from jax import lax
from jax.experimental import pallas as pl
from jax.experimental.pallas import tpu as pltpu
from jax.experimental.pallas import tpu_sc as plsc

N_NODES = 10000
N_EDGES = 320000
D_IN = 128
D_HID = 128
N_CLASSES = 40

NCORES = 2
NSUB = 16
NW = NCORES * NSUB
CHUNK = 128
E_PAD = NW * 80 * CHUNK
E_ROWS = E_PAD // CHUNK
ROWS_PER_TILE = E_ROWS // NW
N_PAD = 10240
DSC = 64
DCLS = 64


def _tile_id():
  c = lax.axis_index("c")
  s = lax.axis_index("s")
  return c, s


@functools.lru_cache(maxsize=None)
def _mesh():
  return plsc.VectorSubcoreMesh(
      core_axis_name="c", subcore_axis_name="s",
      num_cores=NCORES, num_subcores=NSUB)


def _deg_body(dst_hbm, out_hbm, idx_v, ones_v, zb_v, acc_sh, sem):
  c, s = _tile_id()
  w = c * NSUB + s

  def fill_ones(i, _):
    ones_v[pl.ds(i * 16, 16)] = jnp.ones((16,), jnp.float32)
    return 0
  lax.fori_loop(0, CHUNK // 16, fill_ones, 0)

  def fill_zb(i, _):
    zb_v[pl.ds(i * 16, 16)] = jnp.zeros((16,), jnp.float32)
    return 0
  lax.fori_loop(0, (N_PAD // NSUB) // 16, fill_zb, 0)

  pltpu.sync_copy(zb_v, acc_sh.at[pl.ds(s * (N_PAD // NSUB), N_PAD // NSUB)])
  pltpu.sync_copy(dst_hbm.at[pl.ds(w * ROWS_PER_TILE, ROWS_PER_TILE)], idx_v)
  plsc.subcore_barrier()

  def body(j, _):
    pltpu.async_copy(ones_v, acc_sh.at[idx_v.at[j]], sem, add=True)
    return 0
  lax.fori_loop(0, ROWS_PER_TILE, body, 0)

  def drain(j, _):
    pltpu.make_async_copy(ones_v, acc_sh.at[idx_v.at[j]], sem).wait()
    return 0
  lax.fori_loop(0, ROWS_PER_TILE, drain, 0)
  plsc.subcore_barrier()

  @pl.when(s == 0)
  def _():
    pltpu.sync_copy(acc_sh, out_hbm.at[c])


@functools.lru_cache(maxsize=None)
def _deg_call():
  return pl.kernel(
      _deg_body,
      out_type=jax.ShapeDtypeStruct((NCORES, N_PAD), jnp.float32),
      mesh=_mesh(),
      scratch_types=[
          pltpu.VMEM((ROWS_PER_TILE, CHUNK), jnp.int32),
          pltpu.VMEM((CHUNK,), jnp.float32),
          pltpu.VMEM((N_PAD // NSUB,), jnp.float32),
          pltpu.VMEM_SHARED((N_PAD,), jnp.float32),
          pltpu.SemaphoreType.DMA,
      ],
  )


NBUF = 8
PREF = 4
DHALF = DSC // 2
ROWS_PER_TILE2 = E_ROWS // NSUB


def _scatter_body(zin_hbm, src_hbm, dst_hbm, glo_hbm, ghi_hbm,
                  outlo_hbm, outhi_hbm,
                  idxs_v, idxd_v, r0, r1, r2, r3, r4, r5, r6, r7,
                  acc_sh, g_sh, semg, sems):
  rows = (r0, r1, r2, r3, r4, r5, r6, r7)
  c, s = _tile_id()

  @pl.when(s == 0)
  def _():
    pltpu.sync_copy(zin_hbm, acc_sh)

  @pl.when((s == 1) & (c == 0))
  def _():
    pltpu.sync_copy(glo_hbm, g_sh)

  @pl.when((s == 1) & (c == 1))
  def _():
    pltpu.sync_copy(ghi_hbm, g_sh)

  pltpu.sync_copy(src_hbm.at[pl.ds(s * ROWS_PER_TILE2, ROWS_PER_TILE2)], idxs_v)
  pltpu.sync_copy(dst_hbm.at[pl.ds(s * ROWS_PER_TILE2, ROWS_PER_TILE2)], idxd_v)
  plsc.subcore_barrier()

  for b in range(PREF):
    pltpu.async_copy(g_sh.at[idxs_v.at[b]], rows[b], semg.at[b])

  def body(k, _):
    for b in range(NBUF):
      j = k * NBUF + b

      @pl.when(j >= PREF)
      def _():
        jo = j - PREF
        pltpu.make_async_copy(
            rows[(b - PREF) % NBUF], acc_sh.at[idxd_v.at[jo]],
            sems.at[(b - PREF) % NBUF]).wait()

      @pl.when(j + PREF < ROWS_PER_TILE2)
      def _():
        jn = j + PREF
        pltpu.async_copy(g_sh.at[idxs_v.at[jn]], rows[(b + PREF) % NBUF],
                         semg.at[(b + PREF) % NBUF])

      pltpu.make_async_copy(g_sh.at[idxs_v.at[j]], rows[b], semg.at[b]).wait()
      pltpu.async_copy(rows[b], acc_sh.at[idxd_v.at[j]], sems.at[b], add=True)
    return 0

  lax.fori_loop(0, ROWS_PER_TILE2 // NBUF, body, 0)
  for b in range(PREF):
    j = ROWS_PER_TILE2 - PREF + b
    pltpu.make_async_copy(rows[j % NBUF], acc_sh.at[idxd_v.at[j]],
                          sems.at[j % NBUF]).wait()
  plsc.subcore_barrier()

  @pl.when((s == 0) & (c == 0))
  def _():
    pltpu.sync_copy(acc_sh, outlo_hbm)

  @pl.when((s == 0) & (c == 1))
  def _():
    pltpu.sync_copy(acc_sh, outhi_hbm)


@functools.lru_cache(maxsize=None)
def _scatter_call():
  return pl.kernel(
      _scatter_body,
      out_type=[jax.ShapeDtypeStruct((N_PAD, DHALF), jnp.float32),
                jax.ShapeDtypeStruct((N_PAD, DHALF), jnp.float32)],
      mesh=_mesh(),
      scratch_types=(
          [pltpu.VMEM((ROWS_PER_TILE2, CHUNK), jnp.int32),
           pltpu.VMEM((ROWS_PER_TILE2, CHUNK), jnp.int32)]
          + [pltpu.VMEM((CHUNK, DHALF), jnp.float32) for _ in range(NBUF)]
          + [pltpu.VMEM_SHARED((N_PAD, DHALF), jnp.float32),
             pltpu.VMEM_SHARED((N_NODES, DHALF), jnp.float32),
             pltpu.SemaphoreType.DMA((NBUF,)),
             pltpu.SemaphoreType.DMA((NBUF,))]
      ),
      compiler_params=pltpu.CompilerParams(use_tc_tiling_on_sc=False),
  )


_BLK = 1000
_GRID = N_NODES // _BLK


def _dinv(deg_ref):
  return lax.rsqrt(deg_ref[0] + deg_ref[1] + 1.0)


def _tc1_body(deg_ref, x_ref, w1_ref, g10_ref, g11_ref, g12_ref, g13_ref):
  dinv = _dinv(deg_ref)
  h = jnp.dot(x_ref[...], w1_ref[...], preferred_element_type=jnp.float32)
  g1 = h * dinv
  g10_ref[...] = g1[:, 0 * DHALF:1 * DHALF]
  g11_ref[...] = g1[:, 1 * DHALF:2 * DHALF]
  g12_ref[...] = g1[:, 2 * DHALF:3 * DHALF]
  g13_ref[...] = g1[:, 3 * DHALF:4 * DHALF]


def _tc2_body(deg_ref, a0_ref, a1_ref, a2_ref, a3_ref,
              g10_ref, g11_ref, g12_ref, g13_ref, w2_ref, b1_ref,
              g20_ref, g21_ref):
  dinv = _dinv(deg_ref)
  agg = jnp.concatenate(
      [a0_ref[...] + g10_ref[...], a1_ref[...] + g11_ref[...],
       a2_ref[...] + g12_ref[...], a3_ref[...] + g13_ref[...]], axis=1)
  h = jnp.maximum(agg * dinv + b1_ref[...], 0.0)
  g2 = jnp.dot(h, w2_ref[...], preferred_element_type=jnp.float32) * dinv
  g20_ref[...] = g2[:, :DHALF]
  g21_ref[...] = g2[:, DHALF:]


def _tc3_body(deg_ref, a0_ref, a1_ref, g20_ref, g21_ref, b2_ref, out_ref):
  dinv = _dinv(deg_ref)
  t = jnp.concatenate(
      [a0_ref[...] + g20_ref[...], a1_ref[...] + g21_ref[...]], axis=1)
  t = t * dinv + b2_ref[...]
  col = lax.broadcasted_iota(jnp.int32, (_BLK, DCLS), 1)
  valid = col < N_CLASSES
  tm = jnp.where(valid, t, -1e30)
  m = jnp.max(tm, axis=1, keepdims=True)
  e = jnp.where(valid, jnp.exp(tm - m), 0.0)
  ssum = jnp.sum(e, axis=1, keepdims=True)
  out_ref[...] = ((t - m) - jnp.log(ssum))[:, :N_CLASSES]


def _row_spec(d):
  return pl.BlockSpec((_BLK, d), lambda i: (i, 0))


def _deg_spec():
  return pl.BlockSpec((NCORES, _BLK, 1), lambda i: (0, i, 0))


def _full_spec(a, b):
  return pl.BlockSpec((a, b), lambda i: (0, 0))


_tc1_call = pl.pallas_call(
    _tc1_body,
    grid=(_GRID,),
    in_specs=[_deg_spec(), _row_spec(D_IN), _full_spec(D_IN, D_HID)],
    out_specs=[_row_spec(DHALF)] * 4,
    out_shape=[jax.ShapeDtypeStruct((N_NODES, DHALF), jnp.float32)] * 4,
)

_tc2_call = pl.pallas_call(
    _tc2_body,
    grid=(_GRID,),
    in_specs=[_deg_spec()] + [_row_spec(DHALF)] * 8
             + [_full_spec(D_HID, DCLS), _full_spec(1, D_HID)],
    out_specs=[_row_spec(DHALF)] * 2,
    out_shape=[jax.ShapeDtypeStruct((N_NODES, DHALF), jnp.float32)] * 2,
)

_tc3_call = pl.pallas_call(
    _tc3_body,
    grid=(_GRID,),
    in_specs=[_deg_spec()] + [_row_spec(DHALF)] * 4 + [_full_spec(1, DCLS)],
    out_specs=_row_spec(N_CLASSES),
    out_shape=jax.ShapeDtypeStruct((N_NODES, N_CLASSES), jnp.float32),
)


@jax.jit
def kernel(x, edge_index, W1, b1, W2, b2):
  src = edge_index[0].astype(jnp.int32)
  dst = edge_index[1].astype(jnp.int32)
  pad = E_PAD - N_EDGES
  pad_dst = N_NODES + (jnp.arange(pad, dtype=jnp.int32) % (N_PAD - N_NODES))
  src2d = jnp.concatenate([src, jnp.zeros((pad,), jnp.int32)]).reshape(E_ROWS, CHUNK)
  dst2d = jnp.concatenate([dst, pad_dst]).reshape(E_ROWS, CHUNK)

  W2p = jnp.zeros((D_HID, DCLS), jnp.float32).at[:, :N_CLASSES].set(W2)
  b1r = b1.reshape(1, D_HID)
  b2r = jnp.zeros((1, DCLS), jnp.float32).at[0, :N_CLASSES].set(b2)
  zin = jnp.zeros((N_PAD, DHALF), jnp.float32)

  deg_part = _deg_call()(dst2d)
  deg3 = deg_part.reshape(NCORES, N_PAD, 1)

  g10, g11, g12, g13 = _tc1_call(deg3, x, W1)
  scat = _scatter_call()
  a0, a1 = scat(zin, src2d, dst2d, g10, g11)
  a2, a3 = scat(zin, src2d, dst2d, g12, g13)
  g20, g21 = _tc2_call(deg3, a0, a1, a2, a3,
                       g10, g11, g12, g13, W2p, b1r)
  b0, b1_ = scat(zin, src2d, dst2d, g20, g21)
  return _tc3_call(deg3, b0, b1_, g20, g21, b2r)


# speedup vs baseline: 29.1397x; 1.0110x over previous
import functools

import jax
import jax.numpy as jnp
from jax import lax
from jax.experimental import pallas as pl
from jax.experimental.pallas import tpu as pltpu
from jax.experimental.pallas import tpu_sc as plsc

N_NODES = 10000
N_EDGES = 320000
D_IN = 128
D_HID = 128
N_CLASSES = 40

NCORES = 2
NSUB = 16
NW = NCORES * NSUB
CHUNK = 128
E_PAD = NW * 80 * CHUNK
E_ROWS = E_PAD // CHUNK
ROWS_PER_TILE = E_ROWS // NW
N_PAD = 10240
DSC = 64
DCLS = 64


def _tile_id():
  c = lax.axis_index("c")
  s = lax.axis_index("s")
  return c, s


@functools.lru_cache(maxsize=None)
def _mesh():
  return plsc.VectorSubcoreMesh(
      core_axis_name="c", subcore_axis_name="s",
      num_cores=NCORES, num_subcores=NSUB)


def _deg_body(dst_hbm, out_hbm, idx_v, ones_v, zb_v, acc_sh, sem):
  c, s = _tile_id()
  w = c * NSUB + s

  def fill_ones(i, _):
    ones_v[pl.ds(i * 16, 16)] = jnp.ones((16,), jnp.float32)
    return 0
  lax.fori_loop(0, CHUNK // 16, fill_ones, 0)

  def fill_zb(i, _):
    zb_v[pl.ds(i * 16, 16)] = jnp.zeros((16,), jnp.float32)
    return 0
  lax.fori_loop(0, (N_PAD // NSUB) // 16, fill_zb, 0)

  pltpu.sync_copy(zb_v, acc_sh.at[pl.ds(s * (N_PAD // NSUB), N_PAD // NSUB)])
  pltpu.sync_copy(dst_hbm.at[pl.ds(w * ROWS_PER_TILE, ROWS_PER_TILE)], idx_v)
  plsc.subcore_barrier()

  def body(j, _):
    pltpu.async_copy(ones_v, acc_sh.at[idx_v.at[j]], sem, add=True)
    return 0
  lax.fori_loop(0, ROWS_PER_TILE, body, 0)

  def drain(j, _):
    pltpu.make_async_copy(ones_v, acc_sh.at[idx_v.at[j]], sem).wait()
    return 0
  lax.fori_loop(0, ROWS_PER_TILE, drain, 0)
  plsc.subcore_barrier()

  @pl.when(s == 0)
  def _():
    pltpu.sync_copy(acc_sh, out_hbm.at[c])


@functools.lru_cache(maxsize=None)
def _deg_call():
  return pl.kernel(
      _deg_body,
      out_type=jax.ShapeDtypeStruct((NCORES, N_PAD), jnp.float32),
      mesh=_mesh(),
      scratch_types=[
          pltpu.VMEM((ROWS_PER_TILE, CHUNK), jnp.int32),
          pltpu.VMEM((CHUNK,), jnp.float32),
          pltpu.VMEM((N_PAD // NSUB,), jnp.float32),
          pltpu.VMEM_SHARED((N_PAD,), jnp.float32),
          pltpu.SemaphoreType.DMA,
      ],
  )


NBUF = 8
PREF = 4
DHALF = DSC // 2
ROWS_PER_TILE2 = E_ROWS // NSUB


def _scatter_body(zin_hbm, src_hbm, dst_hbm, glo_hbm, ghi_hbm,
                  outlo_hbm, outhi_hbm,
                  idxs_v, idxd_v, r0, r1, r2, r3, r4, r5, r6, r7,
                  acc_sh, g_sh, semg, sems):
  rows = (r0, r1, r2, r3, r4, r5, r6, r7)
  c, s = _tile_id()

  @pl.when(s == 0)
  def _():
    pltpu.sync_copy(zin_hbm, acc_sh)

  @pl.when((s == 1) & (c == 0))
  def _():
    pltpu.sync_copy(glo_hbm, g_sh)

  @pl.when((s == 1) & (c == 1))
  def _():
    pltpu.sync_copy(ghi_hbm, g_sh)

  pltpu.sync_copy(src_hbm.at[pl.ds(s * ROWS_PER_TILE2, ROWS_PER_TILE2)], idxs_v)
  pltpu.sync_copy(dst_hbm.at[pl.ds(s * ROWS_PER_TILE2, ROWS_PER_TILE2)], idxd_v)
  plsc.subcore_barrier()

  for b in range(PREF):
    pltpu.async_copy(g_sh.at[idxs_v.at[b]], rows[b], semg.at[b])

  def body(k, _):
    for b in range(NBUF):
      j = k * NBUF + b

      @pl.when(j >= PREF)
      def _():
        jo = j - PREF
        pltpu.make_async_copy(
            rows[(b - PREF) % NBUF], acc_sh.at[idxd_v.at[jo]],
            sems.at[(b - PREF) % NBUF]).wait()

      @pl.when(j + PREF < ROWS_PER_TILE2)
      def _():
        jn = j + PREF
        pltpu.async_copy(g_sh.at[idxs_v.at[jn]], rows[(b + PREF) % NBUF],
                         semg.at[(b + PREF) % NBUF])

      pltpu.make_async_copy(g_sh.at[idxs_v.at[j]], rows[b], semg.at[b]).wait()
      pltpu.async_copy(rows[b], acc_sh.at[idxd_v.at[j]], sems.at[b], add=True)
    return 0

  lax.fori_loop(0, ROWS_PER_TILE2 // NBUF, body, 0)
  for b in range(PREF):
    j = ROWS_PER_TILE2 - PREF + b
    pltpu.make_async_copy(rows[j % NBUF], acc_sh.at[idxd_v.at[j]],
                          sems.at[j % NBUF]).wait()
  plsc.subcore_barrier()

  @pl.when((s == 0) & (c == 0))
  def _():
    pltpu.sync_copy(acc_sh, outlo_hbm)

  @pl.when((s == 0) & (c == 1))
  def _():
    pltpu.sync_copy(acc_sh, outhi_hbm)


@functools.lru_cache(maxsize=None)
def _scatter_call():
  return pl.kernel(
      _scatter_body,
      out_type=[jax.ShapeDtypeStruct((N_PAD, DHALF), jnp.float32),
                jax.ShapeDtypeStruct((N_PAD, DHALF), jnp.float32)],
      mesh=_mesh(),
      scratch_types=(
          [pltpu.VMEM((ROWS_PER_TILE2, CHUNK), jnp.int32),
           pltpu.VMEM((ROWS_PER_TILE2, CHUNK), jnp.int32)]
          + [pltpu.VMEM((CHUNK, DHALF), jnp.float32) for _ in range(NBUF)]
          + [pltpu.VMEM_SHARED((N_PAD, DHALF), jnp.float32),
             pltpu.VMEM_SHARED((N_NODES, DHALF), jnp.float32),
             pltpu.SemaphoreType.DMA((NBUF,)),
             pltpu.SemaphoreType.DMA((NBUF,))]
      ),
      compiler_params=pltpu.CompilerParams(use_tc_tiling_on_sc=False),
  )


_BLK = 2000
_GRID = N_NODES // _BLK


def _dinv(deg_ref):
  return lax.rsqrt(deg_ref[0] + deg_ref[1] + 1.0)


def _tc1_body(deg_ref, x_ref, w1_ref, g10_ref, g11_ref, g12_ref, g13_ref):
  dinv = _dinv(deg_ref)
  h = jnp.dot(x_ref[...], w1_ref[...], preferred_element_type=jnp.float32)
  g1 = h * dinv
  g10_ref[...] = g1[:, 0 * DHALF:1 * DHALF]
  g11_ref[...] = g1[:, 1 * DHALF:2 * DHALF]
  g12_ref[...] = g1[:, 2 * DHALF:3 * DHALF]
  g13_ref[...] = g1[:, 3 * DHALF:4 * DHALF]


def _tc2_body(deg_ref, a0_ref, a1_ref, a2_ref, a3_ref,
              g10_ref, g11_ref, g12_ref, g13_ref, w2_ref, b1_ref,
              g20_ref, g21_ref):
  dinv = _dinv(deg_ref)
  agg = jnp.concatenate(
      [a0_ref[...] + g10_ref[...], a1_ref[...] + g11_ref[...],
       a2_ref[...] + g12_ref[...], a3_ref[...] + g13_ref[...]], axis=1)
  h = jnp.maximum(agg * dinv + b1_ref[...], 0.0)
  g2 = jnp.dot(h, w2_ref[...], preferred_element_type=jnp.float32) * dinv
  g20_ref[...] = g2[:, :DHALF]
  g21_ref[...] = g2[:, DHALF:]


def _tc3_body(deg_ref, a0_ref, a1_ref, g20_ref, g21_ref, b2_ref, out_ref):
  dinv = _dinv(deg_ref)
  t = jnp.concatenate(
      [a0_ref[...] + g20_ref[...], a1_ref[...] + g21_ref[...]], axis=1)
  t = t * dinv + b2_ref[...]
  col = lax.broadcasted_iota(jnp.int32, (_BLK, DCLS), 1)
  valid = col < N_CLASSES
  tm = jnp.where(valid, t, -1e30)
  m = jnp.max(tm, axis=1, keepdims=True)
  e = jnp.where(valid, jnp.exp(tm - m), 0.0)
  ssum = jnp.sum(e, axis=1, keepdims=True)
  out_ref[...] = ((t - m) - jnp.log(ssum))[:, :N_CLASSES]


def _row_spec(d):
  return pl.BlockSpec((_BLK, d), lambda i: (i, 0))


def _deg_spec():
  return pl.BlockSpec((NCORES, _BLK, 1), lambda i: (0, i, 0))


def _full_spec(a, b):
  return pl.BlockSpec((a, b), lambda i: (0, 0))


_tc1_call = pl.pallas_call(
    _tc1_body,
    grid=(_GRID,),
    in_specs=[_deg_spec(), _row_spec(D_IN), _full_spec(D_IN, D_HID)],
    out_specs=[_row_spec(DHALF)] * 4,
    out_shape=[jax.ShapeDtypeStruct((N_NODES, DHALF), jnp.float32)] * 4,
)

_tc2_call = pl.pallas_call(
    _tc2_body,
    grid=(_GRID,),
    in_specs=[_deg_spec()] + [_row_spec(DHALF)] * 8
             + [_full_spec(D_HID, DCLS), _full_spec(1, D_HID)],
    out_specs=[_row_spec(DHALF)] * 2,
    out_shape=[jax.ShapeDtypeStruct((N_NODES, DHALF), jnp.float32)] * 2,
)

_tc3_call = pl.pallas_call(
    _tc3_body,
    grid=(_GRID,),
    in_specs=[_deg_spec()] + [_row_spec(DHALF)] * 4 + [_full_spec(1, DCLS)],
    out_specs=_row_spec(N_CLASSES),
    out_shape=jax.ShapeDtypeStruct((N_NODES, N_CLASSES), jnp.float32),
)


@jax.jit
def kernel(x, edge_index, W1, b1, W2, b2):
  src = edge_index[0].astype(jnp.int32)
  dst = edge_index[1].astype(jnp.int32)
  pad = E_PAD - N_EDGES
  pad_dst = N_NODES + (jnp.arange(pad, dtype=jnp.int32) % (N_PAD - N_NODES))
  src2d = jnp.concatenate([src, jnp.zeros((pad,), jnp.int32)]).reshape(E_ROWS, CHUNK)
  dst2d = jnp.concatenate([dst, pad_dst]).reshape(E_ROWS, CHUNK)

  W2p = jnp.zeros((D_HID, DCLS), jnp.float32).at[:, :N_CLASSES].set(W2)
  b1r = b1.reshape(1, D_HID)
  b2r = jnp.zeros((1, DCLS), jnp.float32).at[0, :N_CLASSES].set(b2)
  zin = jnp.zeros((N_PAD, DHALF), jnp.float32)

  deg_part = _deg_call()(dst2d)
  deg3 = deg_part.reshape(NCORES, N_PAD, 1)

  g10, g11, g12, g13 = _tc1_call(deg3, x, W1)
  scat = _scatter_call()
  a0, a1 = scat(zin, src2d, dst2d, g10, g11)
  a2, a3 = scat(zin, src2d, dst2d, g12, g13)
  g20, g21 = _tc2_call(deg3, a0, a1, a2, a3,
                       g10, g11, g12, g13, W2p, b1r)
  b0, b1_ = scat(zin, src2d, dst2d, g20, g21)
  return _tc3_call(deg3, b0, b1_, g20, g21, b2r)
